# baseline (speedup 1.0000x reference)
import jax
import jax.numpy as jnp
from jax import lax
from jax.experimental import pallas as pl
from jax.experimental.pallas import tpu as pltpu
from jax.experimental.pallas import tpu_sc as plsc

N_NODES = 10000
N_EDGES = 320000
DIM = 128

NC = 2
NS = 16
NW = NC * NS
NPAD = 10240
EPAD = 327680
EB = 128
NBLK = EPAD // EB
NB0 = 144
NB1 = 160 - NB0
DB = NBLK // NS
NBC = 24
NSPLIT = NS * NB0
NBPAD = NBLK + NB0
RPT = NPAD // NS
RC = 128
NRC = RPT // RC
V = 16

_MESH = plsc.VectorSubcoreMesh(
    core_axis_name="c", subcore_axis_name="s", num_cores=NC, num_subcores=NS)
_SC_PARAMS = pltpu.CompilerParams(needs_layout_passes=False)


def _rsqrt16(v):
    bits = plsc.bitcast(v, jnp.int32)
    y = plsc.bitcast(jnp.int32(0x5F3759DF) - (bits >> 1), jnp.float32)
    for _ in range(3):
        y = y * (1.5 - 0.5 * v * y * y)
    return y


def _deg_body(src_hbm, dst_hbm, ns_hbm, nd_hbm, hist_hbm,
              sbuf, dbuf, hist_s, hist_d, rowbuf, accv):
    c = lax.axis_index("c")
    s = lax.axis_index("s")

    def fz(i, carry):
        hist_s[pl.ds(i * V, V)] = jnp.zeros((V,), jnp.float32)
        hist_d[pl.ds(i * V, V)] = jnp.zeros((V,), jnp.float32)
        return carry

    lax.fori_loop(0, NPAD // V, fz, 0)

    pltpu.sync_copy(src_hbm.at[pl.ds(s * DB, DB)], sbuf)
    pltpu.sync_copy(dst_hbm.at[pl.ds(s * DB, DB)], dbuf)

    ones = jnp.full((V,), 1.0, jnp.float32)

    def hl(j, carry):
        for k in range(EB // V):
            plsc.addupdate_scatter(
                hist_s, [sbuf[j, pl.ds(k * V, V)]], ones)
            plsc.addupdate_scatter(
                hist_d, [dbuf[j, pl.ds(k * V, V)]], ones)
        return carry

    lax.fori_loop(0, DB, hl, 0)

    pltpu.sync_copy(hist_s, hist_hbm.at[c, 0, s])
    pltpu.sync_copy(hist_d, hist_hbm.at[c, 1, s])
    plsc.subcore_barrier()

    for kind, out in ((0, ns_hbm), (1, nd_hbm)):
        pltpu.sync_copy(hist_hbm.at[c, kind, 0, pl.ds(s * RPT, RPT)], accv)
        for r in range(1, NS):
            pltpu.sync_copy(hist_hbm.at[c, kind, r, pl.ds(s * RPT, RPT)],
                            rowbuf)

            def acc_add(i, carry):
                sl = pl.ds(i * V, V)
                accv[sl] = accv[sl] + rowbuf[sl]
                return carry

            lax.fori_loop(0, RPT // V, acc_add, 0)

        def to_norm(i, carry):
            sl = pl.ds(i * V, V)
            accv[sl] = _rsqrt16(jnp.maximum(accv[sl], 1.0))
            return carry

        lax.fori_loop(0, RPT // V, to_norm, 0)
        pltpu.sync_copy(accv, out.at[pl.ds(s * RPT, RPT)])


_deg_call = pl.kernel(
    _deg_body,
    out_type=(jax.ShapeDtypeStruct((NPAD,), jnp.float32),
              jax.ShapeDtypeStruct((NPAD,), jnp.float32),
              jax.ShapeDtypeStruct((NC, 2, NS, NPAD), jnp.float32)),
    mesh=_MESH,
    compiler_params=_SC_PARAMS,
    scratch_types=[
        pltpu.VMEM((DB, EB), jnp.int32),
        pltpu.VMEM((DB, EB), jnp.int32),
        pltpu.VMEM((NPAD,), jnp.float32),
        pltpu.VMEM((NPAD,), jnp.float32),
        pltpu.VMEM((RPT,), jnp.float32),
        pltpu.VMEM((RPT,), jnp.float32),
    ],
)


def _spmm_body(x_hbm, src_hbm, dst_hbm, ns_hbm, nd_hbm, zeros_hbm,
               xs_hbm, out_hbm, idx_s, idx_d, rows, rows2, nsb, ndb, acc,
               gsem0, gsem1):
    c = lax.axis_index("c")
    s = lax.axis_index("s")
    base = s * RPT
    xs_c = xs_hbm.at[c]
    start = jnp.where(c == 0, s * NB0, NSPLIT + s * NB1)
    nblk = jnp.where(c == 0, NB0, NB1)

    pltpu.sync_copy(ns_hbm.at[pl.ds(base, RPT)], nsb)
    pltpu.sync_copy(nd_hbm.at[pl.ds(base, RPT)], ndb)
    pltpu.sync_copy(zeros_hbm, acc.at[pl.ds(base, RPT)])


    rows_rc = rows.at[pl.ds(0, RC)]

    def _scale_rows(norm_ref, chunk):
        def scale(g, carry):
            nv = norm_ref[pl.ds(chunk * RC + g * V, V)]
            for r16 in range(V):
                w = nv[r16]
                for k in range(DIM // V):
                    sl = pl.ds(k * V, V)
                    rows_rc[g * V + r16, sl] = rows_rc[g * V + r16, sl] * w
            return carry

        lax.fori_loop(0, RC // V, scale, 0)
    for chunk in range(NRC):
        r0 = base + chunk * RC
        pltpu.sync_copy(x_hbm.at[pl.ds(r0, RC)], rows_rc)
        _scale_rows(nsb, chunk)
        pltpu.sync_copy(rows_rc, xs_c.at[pl.ds(r0, RC)])

    plsc.subcore_barrier()

    def gstart(j, buf, sem):
        pltpu.async_copy(xs_c.at[idx_s.at[j]], buf, sem)

    def gwait(buf, sem):
        pltpu.make_async_copy(x_hbm.at[pl.ds(0, EB)], buf, sem).wait()

    for ci in range(NB0 // NBC):
        cstart = ci * NBC
        nhere = jnp.clip(nblk - cstart, 0, NBC)
        pltpu.sync_copy(src_hbm.at[pl.ds(start + cstart, NBC)], idx_s)
        pltpu.sync_copy(dst_hbm.at[pl.ds(start + cstart, NBC)], idx_d)

        @pl.when(nhere >= 2)
        def _():
            gstart(0, rows, gsem0)
            gstart(1, rows2, gsem1)

            def blk2(i, carry):
                j0 = 2 * i
                gwait(rows, gsem0)
                pltpu.sync_copy(rows, acc.at[idx_d.at[j0]], add=True)
                gstart(j0 + 2, rows, gsem0)
                gwait(rows2, gsem1)
                pltpu.sync_copy(rows2, acc.at[idx_d.at[j0 + 1]], add=True)
                gstart(j0 + 3, rows2, gsem1)
                return carry

            lax.fori_loop(0, nhere // 2 - 1, blk2, 0)
            gwait(rows, gsem0)
            pltpu.sync_copy(rows, acc.at[idx_d.at[nhere - 2]], add=True)
            gwait(rows2, gsem1)
            pltpu.sync_copy(rows2, acc.at[idx_d.at[nhere - 1]], add=True)
    plsc.subcore_barrier()

    for chunk in range(NRC):
        r0 = base + chunk * RC
        pltpu.sync_copy(acc.at[pl.ds(r0, RC)], rows_rc)
        _scale_rows(ndb, chunk)
        pltpu.sync_copy(rows_rc, out_hbm.at[c, pl.ds(r0, RC)])


_spmm_call = pl.kernel(
    _spmm_body,
    out_type=(jax.ShapeDtypeStruct((NC, NPAD, DIM), jnp.float32),
              jax.ShapeDtypeStruct((NC, NPAD, DIM), jnp.float32)),
    mesh=_MESH,
    compiler_params=_SC_PARAMS,
    scratch_types=[
        pltpu.VMEM((NBC, EB), jnp.int32),
        pltpu.VMEM((NBC, EB), jnp.int32),
        pltpu.VMEM((EB, DIM), jnp.float32),
        pltpu.VMEM((EB, DIM), jnp.float32),
        pltpu.VMEM((RPT,), jnp.float32),
        pltpu.VMEM((RPT,), jnp.float32),
        pltpu.VMEM_SHARED((NPAD, DIM), jnp.float32),
        pltpu.SemaphoreType.DMA,
        pltpu.SemaphoreType.DMA,
    ],
)


BR = 1024
GRID = NPAD // BR

_row_spec = pl.BlockSpec((BR, DIM), lambda i: (i, 0))
_mat_spec = pl.BlockSpec((DIM, DIM), lambda i: (0, 0))
_bias_spec = pl.BlockSpec((1, DIM), lambda i: (0, 0))


def _tc_body(p0, p1, b_ref, w_ref, o_ref):
    agg = p0[...] + p1[...]
    o_ref[...] = jnp.maximum(
        jnp.dot(agg, w_ref[...], preferred_element_type=jnp.float32)
        + b_ref[...], 0.0)


_tc_call = pl.pallas_call(
    _tc_body, grid=(GRID,),
    in_specs=[_row_spec, _row_spec, _bias_spec, _mat_spec],
    out_specs=_row_spec,
    out_shape=jax.ShapeDtypeStruct((NPAD, DIM), jnp.float32))


@jax.jit
def kernel(edge_index, node_feature, W1, b1, W2, b2):
    pad = jnp.full((NBPAD * EB - N_EDGES,), N_NODES, dtype=jnp.int32)
    srcp = jnp.concatenate(
        [edge_index[0].astype(jnp.int32), pad]).reshape(NBPAD, EB)
    dstp = jnp.concatenate(
        [edge_index[1].astype(jnp.int32), pad]).reshape(NBPAD, EB)
    x_pad = jnp.pad(node_feature, ((0, NPAD - N_NODES), (0, 0)))
    zeros_row = jnp.zeros((RPT, DIM), jnp.float32)

    ns, nd, _ = _deg_call(srcp, dstp)
    _, p = _spmm_call(x_pad, srcp, dstp, ns, nd, zeros_row)
    y1 = _tc_call(p[0], p[1], b1[None, :], W1)
    _, q = _spmm_call(y1, srcp, dstp, ns, nd, zeros_row)
    out = _tc_call(q[0], q[1], b2[None, :], W2)
    return out[:N_NODES]


# speedup vs baseline: 1.0307x; 1.0307x over previous
import jax
import jax.numpy as jnp
from jax import lax
from jax.experimental import pallas as pl
from jax.experimental.pallas import tpu as pltpu
from jax.experimental.pallas import tpu_sc as plsc

N_NODES = 10000
N_EDGES = 320000
DIM = 128

NC = 2
NS = 16
NW = NC * NS
NPAD = 10240
EPAD = 327680
EB = 128
NBLK = EPAD // EB
NB0 = 144
NB1 = 160 - NB0
DB = NBLK // NS
NBC = 48
NSPLIT = NS * NB0
NBPAD = NBLK + NB0
RPT = NPAD // NS
RC = 128
NRC = RPT // RC
V = 16

_MESH = plsc.VectorSubcoreMesh(
    core_axis_name="c", subcore_axis_name="s", num_cores=NC, num_subcores=NS)
_SC_PARAMS = pltpu.CompilerParams(needs_layout_passes=False)


def _rsqrt16(v):
    bits = plsc.bitcast(v, jnp.int32)
    y = plsc.bitcast(jnp.int32(0x5F3759DF) - (bits >> 1), jnp.float32)
    for _ in range(3):
        y = y * (1.5 - 0.5 * v * y * y)
    return y


def _deg_body(cat_hbm, norm_hbm, hist_hbm, sbuf, hist, rowbuf, accv):
    c = lax.axis_index("c")
    s = lax.axis_index("s")

    def fz(i, carry):
        hist[pl.ds(i * V, V)] = jnp.zeros((V,), jnp.float32)
        return carry

    lax.fori_loop(0, NPAD // V, fz, 0)

    pltpu.sync_copy(cat_hbm.at[pl.ds(c * NBPAD + s * DB, DB)], sbuf)

    ones = jnp.full((V,), 1.0, jnp.float32)

    def hl(j, carry):
        for k in range(EB // V):
            plsc.addupdate_scatter(
                hist, [sbuf[j, pl.ds(k * V, V)]], ones)
        return carry

    lax.fori_loop(0, DB, hl, 0)

    pltpu.sync_copy(hist, hist_hbm.at[c, s])
    plsc.subcore_barrier()

    pltpu.sync_copy(hist_hbm.at[c, 0, pl.ds(s * RPT, RPT)], accv)
    for r in range(1, NS):
        pltpu.sync_copy(hist_hbm.at[c, r, pl.ds(s * RPT, RPT)], rowbuf)

        def acc_add(i, carry):
            sl = pl.ds(i * V, V)
            accv[sl] = accv[sl] + rowbuf[sl]
            return carry

        lax.fori_loop(0, RPT // V, acc_add, 0)

    def to_norm(i, carry):
        sl = pl.ds(i * V, V)
        accv[sl] = _rsqrt16(jnp.maximum(accv[sl], 1.0))
        return carry

    lax.fori_loop(0, RPT // V, to_norm, 0)
    pltpu.sync_copy(accv, norm_hbm.at[c, pl.ds(s * RPT, RPT)])


_deg_call = pl.kernel(
    _deg_body,
    out_type=(jax.ShapeDtypeStruct((2, NPAD), jnp.float32),
              jax.ShapeDtypeStruct((NC, NS, NPAD), jnp.float32)),
    mesh=_MESH,
    compiler_params=_SC_PARAMS,
    scratch_types=[
        pltpu.VMEM((DB, EB), jnp.int32),
        pltpu.VMEM((NPAD,), jnp.float32),
        pltpu.VMEM((RPT,), jnp.float32),
        pltpu.VMEM((RPT,), jnp.float32),
    ],
)


def _spmm_body(x_hbm, src_hbm, dst_hbm, ns_hbm, nd_hbm, zeros_hbm,
               xs_hbm, out_hbm, idx_s, idx_d, rows, nsb, ndb, acc):
    c = lax.axis_index("c")
    s = lax.axis_index("s")
    base = s * RPT
    xs_c = xs_hbm.at[c]
    start = jnp.where(c == 0, s * NB0, NSPLIT + s * NB1)
    nblk = jnp.where(c == 0, NB0, NB1)

    pltpu.sync_copy(ns_hbm.at[pl.ds(base, RPT)], nsb)
    pltpu.sync_copy(nd_hbm.at[pl.ds(base, RPT)], ndb)
    pltpu.sync_copy(zeros_hbm, acc.at[pl.ds(base, RPT)])


    rows_rc = rows.at[pl.ds(0, RC)]

    def _scale_rows(norm_ref, chunk):
        def scale(g, carry):
            nv = norm_ref[pl.ds(chunk * RC + g * V, V)]
            for r16 in range(V):
                w = nv[r16]
                for k in range(DIM // V):
                    sl = pl.ds(k * V, V)
                    rows_rc[g * V + r16, sl] = rows_rc[g * V + r16, sl] * w
            return carry

        lax.fori_loop(0, RC // V, scale, 0)
    for chunk in range(NRC):
        r0 = base + chunk * RC
        pltpu.sync_copy(x_hbm.at[pl.ds(r0, RC)], rows_rc)
        _scale_rows(nsb, chunk)
        pltpu.sync_copy(rows_rc, xs_c.at[pl.ds(r0, RC)])

    plsc.subcore_barrier()

    def blk(j, carry):
        pltpu.sync_copy(xs_c.at[idx_s.at[j]], rows)
        pltpu.sync_copy(rows, acc.at[idx_d.at[j]], add=True)
        return carry

    for ci in range(NB0 // NBC):
        cstart = ci * NBC
        pltpu.sync_copy(src_hbm.at[pl.ds(start + cstart, NBC)], idx_s)
        pltpu.sync_copy(dst_hbm.at[pl.ds(start + cstart, NBC)], idx_d)
        lax.fori_loop(0, jnp.clip(nblk - cstart, 0, NBC), blk, 0)
    plsc.subcore_barrier()

    for chunk in range(NRC):
        r0 = base + chunk * RC
        pltpu.sync_copy(acc.at[pl.ds(r0, RC)], rows_rc)
        _scale_rows(ndb, chunk)
        pltpu.sync_copy(rows_rc, out_hbm.at[c, pl.ds(r0, RC)])


_spmm_call = pl.kernel(
    _spmm_body,
    out_type=(jax.ShapeDtypeStruct((NC, NPAD, DIM), jnp.float32),
              jax.ShapeDtypeStruct((NC, NPAD, DIM), jnp.float32)),
    mesh=_MESH,
    compiler_params=_SC_PARAMS,
    scratch_types=[
        pltpu.VMEM((NBC, EB), jnp.int32),
        pltpu.VMEM((NBC, EB), jnp.int32),
        pltpu.VMEM((EB, DIM), jnp.float32),
        pltpu.VMEM((RPT,), jnp.float32),
        pltpu.VMEM((RPT,), jnp.float32),
        pltpu.VMEM_SHARED((NPAD, DIM), jnp.float32),
    ],
)


BR = 1024
GRID = NPAD // BR

_row_spec = pl.BlockSpec((BR, DIM), lambda i: (i, 0))
_mat_spec = pl.BlockSpec((DIM, DIM), lambda i: (0, 0))
_bias_spec = pl.BlockSpec((1, DIM), lambda i: (0, 0))


def _tc_body(p0, p1, b_ref, w_ref, o_ref):
    agg = p0[...] + p1[...]
    o_ref[...] = jnp.maximum(
        jnp.dot(agg, w_ref[...], preferred_element_type=jnp.float32)
        + b_ref[...], 0.0)


_tc_call = pl.pallas_call(
    _tc_body, grid=(GRID,),
    in_specs=[_row_spec, _row_spec, _bias_spec, _mat_spec],
    out_specs=_row_spec,
    out_shape=jax.ShapeDtypeStruct((NPAD, DIM), jnp.float32))


@jax.jit
def kernel(edge_index, node_feature, W1, b1, W2, b2):
    pad = jnp.full((NBPAD * EB - N_EDGES,), N_NODES, dtype=jnp.int32)
    srcp = jnp.concatenate(
        [edge_index[0].astype(jnp.int32), pad]).reshape(NBPAD, EB)
    dstp = jnp.concatenate(
        [edge_index[1].astype(jnp.int32), pad]).reshape(NBPAD, EB)
    x_pad = jnp.pad(node_feature, ((0, NPAD - N_NODES), (0, 0)))
    zeros_row = jnp.zeros((RPT, DIM), jnp.float32)

    norms, _ = _deg_call(jnp.concatenate([srcp, dstp]))
    ns, nd = norms[0], norms[1]
    _, p = _spmm_call(x_pad, srcp, dstp, ns, nd, zeros_row)
    y1 = _tc_call(p[0], p[1], b1[None, :], W1)
    _, q = _spmm_call(y1, srcp, dstp, ns, nd, zeros_row)
    out = _tc_call(q[0], q[1], b2[None, :], W2)
    return out[:N_NODES]
